# baseline (speedup 1.0000x reference)
import functools

import jax
import jax.numpy as jnp
from jax import lax
from jax.experimental import pallas as pl
from jax.experimental.pallas import tpu as pltpu
from jax.experimental.pallas import tpu_sc as plsc

_L = 16


def _sc_body(nc, rows_w, ch, x_hbm, t_hbm, o_hbm,
             tbA, xbA, obA, tbB, xbB, obB, isA, isB, osA, osB):
    wid = lax.axis_index("s") * nc + lax.axis_index("c")
    base = wid * rows_w
    nchunk = rows_w // ch
    d = t_hbm.shape[1]

    def fire_in(tb, xb, isem, cidx):
        row0 = base + cidx * ch
        pltpu.async_copy(t_hbm.at[pl.ds(row0, ch)], tb, isem)
        pltpu.async_copy(x_hbm.at[:, pl.ds(row0, ch)], xb, isem)

    def wait_in(tb, xb, isem):
        pltpu.make_async_copy(t_hbm.at[pl.ds(0, ch)], tb, isem).wait()
        pltpu.make_async_copy(x_hbm.at[:, pl.ds(0, ch)], xb, isem).wait()

    def fire_out(ob, osem, cidx):
        row0 = base + cidx * ch
        pltpu.async_copy(ob, o_hbm.at[:, pl.ds(row0, ch)], osem)

    def wait_out(ob, osem):
        pltpu.make_async_copy(ob, o_hbm.at[:, pl.ds(0, ch)], osem).wait()

    def compute(tb, xb, ob):
        def row(r, rc):
            for j in range(d // _L):
                sl = pl.ds(j * _L, _L)
                t = tb[r, sl]
                for b in range(4):
                    ob[b, r, sl] = xb[b, r, sl] + t
            return rc

        lax.fori_loop(0, ch, row, 0)

    fire_in(tbA, xbA, isA, 0)

    def body(c, carry):
        even = 2 * c
        fire_in(tbB, xbB, isB, even + 1)
        wait_in(tbA, xbA, isA)

        @pl.when(c > 0)
        def _():
            wait_out(obA, osA)

        compute(tbA, xbA, obA)
        fire_out(obA, osA, even)
        fire_in(tbA, xbA, isA, jnp.minimum(even + 2, nchunk - 2))
        wait_in(tbB, xbB, isB)

        @pl.when(c > 0)
        def _():
            wait_out(obB, osB)

        compute(tbB, xbB, obB)
        fire_out(obB, osB, even + 1)
        return carry

    lax.fori_loop(0, nchunk // 2, body, 0)
    wait_in(tbA, xbA, isA)
    wait_out(obA, osA)
    wait_out(obB, osB)


def _sc_part(inputs, table, k_rows):
    B, S, D = inputs.shape
    info = plsc.get_sparse_core_info()
    nc, ns = info.num_cores, info.num_subcores
    nw = nc * ns
    rows_w = k_rows // nw
    ch = 4

    mesh = plsc.VectorSubcoreMesh(core_axis_name="c", subcore_axis_name="s")
    k = functools.partial(
        pl.kernel,
        mesh=mesh,
        out_type=jax.ShapeDtypeStruct((B, k_rows, D), inputs.dtype),
        scratch_types=[
            pltpu.VMEM((ch, D), jnp.float32),
            pltpu.VMEM((B, ch, D), jnp.float32),
            pltpu.VMEM((B, ch, D), jnp.float32),
            pltpu.VMEM((ch, D), jnp.float32),
            pltpu.VMEM((B, ch, D), jnp.float32),
            pltpu.VMEM((B, ch, D), jnp.float32),
            pltpu.SemaphoreType.DMA,
            pltpu.SemaphoreType.DMA,
            pltpu.SemaphoreType.DMA,
            pltpu.SemaphoreType.DMA,
        ],
    )(functools.partial(_sc_body, nc, rows_w, ch))
    return k(inputs, table)


def _tc_add_body(x_ref, t_ref, o_ref):
    o_ref[...] = x_ref[...] + t_ref[...]


def _tc_part(inputs, table, k_rows):
    B, S, D = inputs.shape
    bs = 512
    off = k_rows // bs
    grid = ((S - k_rows) // bs, B)
    return pl.pallas_call(
        _tc_add_body,
        grid=grid,
        in_specs=[
            pl.BlockSpec((1, bs, D), lambda i, b: (b, i + off, 0)),
            pl.BlockSpec((bs, D), lambda i, b: (i + off, 0)),
        ],
        out_specs=pl.BlockSpec((1, bs, D), lambda i, b: (b, i + off, 0)),
        out_shape=jax.ShapeDtypeStruct((B, S, D), inputs.dtype),
    )(inputs, table)


def kernel(inputs, table):
    k_rows = 2048
    tc_out = _tc_part(inputs, table, k_rows)
    sc_out = _sc_part(inputs, table, k_rows)
    return lax.dynamic_update_slice(tc_out, sc_out, (0, 0, 0))


# speedup vs baseline: 1.1865x; 1.1865x over previous
import functools

import jax
import jax.numpy as jnp
from jax import lax
from jax.experimental import pallas as pl
from jax.experimental.pallas import tpu as pltpu
from jax.experimental.pallas import tpu_sc as plsc

_L = 16


def _sc_body(nc, rows_w, ch, x_hbm, t_hbm, o_hbm,
             tbA, xbA, obA, tbB, xbB, obB, isA, isB, osA, osB):
    wid = lax.axis_index("s") * nc + lax.axis_index("c")
    base = wid * rows_w
    nchunk = rows_w // ch
    d = t_hbm.shape[1]

    def fire_in(tb, xb, isem, cidx):
        row0 = base + cidx * ch
        pltpu.async_copy(t_hbm.at[pl.ds(row0, ch)], tb, isem)
        pltpu.async_copy(x_hbm.at[:, pl.ds(row0, ch)], xb, isem)

    def wait_in(tb, xb, isem):
        pltpu.make_async_copy(t_hbm.at[pl.ds(0, ch)], tb, isem).wait()
        pltpu.make_async_copy(x_hbm.at[:, pl.ds(0, ch)], xb, isem).wait()

    def fire_out(ob, osem, cidx):
        row0 = base + cidx * ch
        pltpu.async_copy(ob, o_hbm.at[:, pl.ds(row0, ch)], osem)

    def wait_out(ob, osem):
        pltpu.make_async_copy(ob, o_hbm.at[:, pl.ds(0, ch)], osem).wait()

    def compute(tb, xb, ob):
        def row(r, rc):
            for j in range(d // _L):
                sl = pl.ds(j * _L, _L)
                t = tb[r, sl]
                for b in range(4):
                    ob[b, r, sl] = xb[b, r, sl] + t
            return rc

        lax.fori_loop(0, ch, row, 0)

    fire_in(tbA, xbA, isA, 0)

    def body(c, carry):
        even = 2 * c
        fire_in(tbB, xbB, isB, even + 1)
        wait_in(tbA, xbA, isA)

        @pl.when(c > 0)
        def _():
            wait_out(obA, osA)

        compute(tbA, xbA, obA)
        fire_out(obA, osA, even)
        fire_in(tbA, xbA, isA, jnp.minimum(even + 2, nchunk - 2))
        wait_in(tbB, xbB, isB)

        @pl.when(c > 0)
        def _():
            wait_out(obB, osB)

        compute(tbB, xbB, obB)
        fire_out(obB, osB, even + 1)
        return carry

    lax.fori_loop(0, nchunk // 2, body, 0)
    wait_in(tbA, xbA, isA)
    wait_out(obA, osA)
    wait_out(obB, osB)


def _sc_part(inputs, table, k_rows):
    B, S, D = inputs.shape
    info = plsc.get_sparse_core_info()
    nc, ns = info.num_cores, info.num_subcores
    nw = nc * ns
    rows_w = k_rows // nw
    ch = 4

    mesh = plsc.VectorSubcoreMesh(core_axis_name="c", subcore_axis_name="s")
    k = functools.partial(
        pl.kernel,
        mesh=mesh,
        out_type=jax.ShapeDtypeStruct((B, k_rows, D), inputs.dtype),
        scratch_types=[
            pltpu.VMEM((ch, D), jnp.float32),
            pltpu.VMEM((B, ch, D), jnp.float32),
            pltpu.VMEM((B, ch, D), jnp.float32),
            pltpu.VMEM((ch, D), jnp.float32),
            pltpu.VMEM((B, ch, D), jnp.float32),
            pltpu.VMEM((B, ch, D), jnp.float32),
            pltpu.SemaphoreType.DMA,
            pltpu.SemaphoreType.DMA,
            pltpu.SemaphoreType.DMA,
            pltpu.SemaphoreType.DMA,
        ],
    )(functools.partial(_sc_body, nc, rows_w, ch))
    return k(inputs, table)


def _tc_add_body(x_ref, t_ref, o_ref):
    o_ref[...] = x_ref[...] + t_ref[...]


def _tc_part(inputs, table, k_rows):
    B, S, D = inputs.shape
    bs = 512
    off = k_rows // bs
    grid = ((S - k_rows) // bs, B)
    return pl.pallas_call(
        _tc_add_body,
        grid=grid,
        in_specs=[
            pl.BlockSpec((1, bs, D), lambda i, b: (b, i + off, 0)),
            pl.BlockSpec((bs, D), lambda i, b: (i + off, 0)),
        ],
        out_specs=pl.BlockSpec((1, bs, D), lambda i, b: (b, i + off, 0)),
        out_shape=jax.ShapeDtypeStruct((B, S, D), inputs.dtype),
    )(inputs, table)


def kernel(inputs, table):
    k_rows = 512
    tc_out = _tc_part(inputs, table, k_rows)
    sc_out = _sc_part(inputs, table, k_rows)
    return lax.dynamic_update_slice(tc_out, sc_out, (0, 0, 0))


# speedup vs baseline: 1.2893x; 1.0866x over previous
import functools

import jax
import jax.numpy as jnp
from jax import lax
from jax.experimental import pallas as pl
from jax.experimental.pallas import tpu as pltpu
from jax.experimental.pallas import tpu_sc as plsc

_L = 16


def _sc_body(nc, rows_w, ch, x_hbm, t_hbm, o_hbm,
             tbA, xbA, obA, tbB, xbB, obB, isA, isB, osA, osB):
    wid = lax.axis_index("s") * nc + lax.axis_index("c")
    base = wid * rows_w
    nchunk = rows_w // ch
    d = t_hbm.shape[1]

    def fire_in(tb, xb, isem, cidx):
        row0 = base + cidx * ch
        pltpu.async_copy(t_hbm.at[pl.ds(row0, ch)], tb, isem)
        pltpu.async_copy(x_hbm.at[:, pl.ds(row0, ch)], xb, isem)

    def wait_in(tb, xb, isem):
        pltpu.make_async_copy(t_hbm.at[pl.ds(0, ch)], tb, isem).wait()
        pltpu.make_async_copy(x_hbm.at[:, pl.ds(0, ch)], xb, isem).wait()

    def fire_out(ob, osem, cidx):
        row0 = base + cidx * ch
        pltpu.async_copy(ob, o_hbm.at[:, pl.ds(row0, ch)], osem)

    def wait_out(ob, osem):
        pltpu.make_async_copy(ob, o_hbm.at[:, pl.ds(0, ch)], osem).wait()

    def compute(tb, xb, ob):
        def row(r, rc):
            for j in range(d // _L):
                sl = pl.ds(j * _L, _L)
                t = tb[r, sl]
                for b in range(4):
                    ob[b, r, sl] = xb[b, r, sl] + t
            return rc

        lax.fori_loop(0, ch, row, 0)

    fire_in(tbA, xbA, isA, 0)

    def body(c, carry):
        even = 2 * c
        fire_in(tbB, xbB, isB, even + 1)
        wait_in(tbA, xbA, isA)

        @pl.when(c > 0)
        def _():
            wait_out(obA, osA)

        compute(tbA, xbA, obA)
        fire_out(obA, osA, even)
        fire_in(tbA, xbA, isA, jnp.minimum(even + 2, nchunk - 2))
        wait_in(tbB, xbB, isB)

        @pl.when(c > 0)
        def _():
            wait_out(obB, osB)

        compute(tbB, xbB, obB)
        fire_out(obB, osB, even + 1)
        return carry

    lax.fori_loop(0, nchunk // 2, body, 0)
    wait_in(tbA, xbA, isA)
    wait_out(obA, osA)
    wait_out(obB, osB)


def _sc_part(inputs, table, k_rows):
    B, S, D = inputs.shape
    info = plsc.get_sparse_core_info()
    nc, ns = info.num_cores, info.num_subcores
    nw = nc * ns
    rows_w = k_rows // nw
    ch = 4

    mesh = plsc.VectorSubcoreMesh(core_axis_name="c", subcore_axis_name="s")
    k = functools.partial(
        pl.kernel,
        mesh=mesh,
        out_type=jax.ShapeDtypeStruct((B, k_rows, D), inputs.dtype),
        scratch_types=[
            pltpu.VMEM((ch, D), jnp.float32),
            pltpu.VMEM((B, ch, D), jnp.float32),
            pltpu.VMEM((B, ch, D), jnp.float32),
            pltpu.VMEM((ch, D), jnp.float32),
            pltpu.VMEM((B, ch, D), jnp.float32),
            pltpu.VMEM((B, ch, D), jnp.float32),
            pltpu.SemaphoreType.DMA,
            pltpu.SemaphoreType.DMA,
            pltpu.SemaphoreType.DMA,
            pltpu.SemaphoreType.DMA,
        ],
    )(functools.partial(_sc_body, nc, rows_w, ch))
    return k(inputs, table)


def _tc_add_body(x_ref, t_ref, o_ref):
    o_ref[...] = x_ref[...] + t_ref[...]


def _tc_part(inputs, table, k_rows):
    B, S, D = inputs.shape
    bs = 512
    off = k_rows // bs
    grid = ((S - k_rows) // bs, B)
    return pl.pallas_call(
        _tc_add_body,
        grid=grid,
        in_specs=[
            pl.BlockSpec((1, bs, D), lambda i, b: (b, i + off, 0)),
            pl.BlockSpec((bs, D), lambda i, b: (i + off, 0)),
        ],
        out_specs=pl.BlockSpec((1, bs, D), lambda i, b: (b, i + off, 0)),
        out_shape=jax.ShapeDtypeStruct((B, S, D), inputs.dtype),
    )(inputs, table)


def kernel(inputs, table):
    k_rows = 256
    tc_out = _tc_part(inputs, table, k_rows)
    sc_out = _sc_part(inputs, table, k_rows)
    return lax.dynamic_update_slice(tc_out, sc_out, (0, 0, 0))
